# Initial kernel scaffold; baseline (speedup 1.0000x reference)
import jax
import jax.numpy as jnp
from jax.experimental import pallas as pl


def kernel(inputs, r_tm1, m_tm1, c_wu_tm1, c_wlu_tm1, c_wr_tm1, h_tm1, c_tm1, write_gate, Wk, Uk, bk):
    raise NotImplementedError("write your pallas kernel here")


# speedup vs baseline: 29.8852x; 29.8852x over previous
import jax
import jax.numpy as jnp
from jax.experimental import pallas as pl
from jax.experimental.pallas import tpu as pltpu

B, D, U, M = 1024, 128, 64, 16384
USAGE_DECAY = 0.95
MB = 512
NB = M // MB


def _lstm_body(ctrl_ref, h_ref, c_ref, wk_ref, uk_ref, bk_ref,
               key_ref, cnew_ref, nkey_ref):
    z = (jax.lax.dot_general(ctrl_ref[...], wk_ref[...],
                             (((1,), (0,)), ((), ())),
                             preferred_element_type=jnp.float32)
         + jax.lax.dot_general(h_ref[...], uk_ref[...],
                               (((1,), (0,)), ((), ())),
                               preferred_element_type=jnp.float32)
         + bk_ref[0:1, :])
    zi = z[:, 0 * U:1 * U]
    zf = z[:, 1 * U:2 * U]
    zc = z[:, 2 * U:3 * U]
    zo = z[:, 3 * U:4 * U]
    i = jax.nn.sigmoid(zi)
    f = jax.nn.sigmoid(zf)
    c_new = f * c_ref[...] + i * jnp.tanh(zc)
    o = jax.nn.sigmoid(zo)
    key = o * jnp.tanh(c_new)
    key_ref[...] = key
    cnew_ref[...] = c_new
    nkey_ref[...] = key / jnp.sqrt(
        jnp.maximum(jnp.sum(key * key, axis=1, keepdims=True), 1e-12))


def _pass1_body(m_ref, cwu1_ref, cwlu1_ref, cwr1_ref, nkey_ref, key_ref,
                wg_ref,
                cwr_ref, cwu_ref, memw_ref, read_ref, cmin_ref, carg_ref):
    step = pl.program_id(0)
    mblk = m_ref[...]
    nm = mblk / jnp.sqrt(
        jnp.maximum(jnp.sum(mblk * mblk, axis=1, keepdims=True), 1e-12))
    cos = jax.lax.dot_general(nm, nkey_ref[...],
                              (((1,), (1,)), ((), ())),
                              preferred_element_type=jnp.float32)
    rowmax = jnp.max(cos, axis=1, keepdims=True)
    e = jnp.exp(cos - rowmax)
    cwr = e / jnp.sum(e, axis=1, keepdims=True)
    cwr_ref[...] = cwr

    wg = wg_ref[0, 0]
    cww = wg * cwr1_ref[...] + (1.0 - wg) + cwlu1_ref[...]
    cwu = USAGE_DECAY * cwu1_ref[...] + cwr + cww
    cwu_ref[...] = cwu

    memw_ref[...] = jax.lax.dot_general(cww, key_ref[...],
                                        (((1,), (0,)), ((), ())),
                                        preferred_element_type=jnp.float32)

    rpart = jax.lax.dot_general(cwr, mblk,
                                (((0,), (0,)), ((), ())),
                                preferred_element_type=jnp.float32)

    blkmin = jnp.min(cwu, axis=0, keepdims=True)
    rows = jax.lax.broadcasted_iota(jnp.int32, (MB, B), 0) + step * MB
    cand = jnp.max(jnp.where(cwu == blkmin, rows, -1),
                   axis=0, keepdims=True)
    blkmin8 = jnp.broadcast_to(blkmin, (8, B))
    cand8 = jnp.broadcast_to(cand, (8, B))

    @pl.when(step == 0)
    def _init():
        read_ref[...] = rpart
        cmin_ref[...] = blkmin8
        carg_ref[...] = cand8

    @pl.when(step != 0)
    def _acc():
        read_ref[...] += rpart
        run_min = cmin_ref[...]
        upd = blkmin8 <= run_min
        carg_ref[...] = jnp.where(upd, cand8, carg_ref[...])
        cmin_ref[...] = jnp.minimum(run_min, blkmin8)


def _pass2_body(cwu_ref, cmin_ref, carg_ref, memw_ref, m_ref,
                cwlu_ref, mem_ref, sel_ref):
    step = pl.program_id(0)

    @pl.when(step == 0)
    def _select():
        cm = cmin_ref[0:1, :]
        minv = jnp.min(cm)
        lane = jax.lax.broadcasted_iota(jnp.int32, (1, B), 1)
        i_nth = jnp.min(jnp.where(cm == minv, lane, 2 ** 30))
        selv = jnp.max(jnp.where(lane == i_nth, carg_ref[0:1, :], -1))
        sel_ref[0] = selv

    cmrow = cmin_ref[0:1, :]
    cwlu_ref[...] = (cwu_ref[...] <= cmrow).astype(jnp.float32)

    sel = sel_ref[0]
    rows = jax.lax.broadcasted_iota(jnp.int32, (MB, 1), 0) + step * MB
    keep = (rows != sel).astype(jnp.float32)
    mem_ref[...] = memw_ref[...] + (keep * float(B)) * m_ref[...]


def kernel(inputs, r_tm1, m_tm1, c_wu_tm1, c_wlu_tm1, c_wr_tm1, h_tm1,
           c_tm1, write_gate, Wk, Uk, bk):
    ctrl_in = jnp.concatenate([inputs, r_tm1], axis=1)
    bk8 = jnp.broadcast_to(bk.reshape(1, 4 * U), (8, 4 * U))
    wg8 = jnp.broadcast_to(jax.nn.sigmoid(write_gate).reshape(1, 1), (8, 128))

    key_list, c_ctrl_new, n_key = pl.pallas_call(
        _lstm_body,
        out_shape=[jax.ShapeDtypeStruct((B, U), jnp.float32)] * 3,
    )(ctrl_in, h_tm1, c_tm1, Wk, Uk, bk8)

    f32 = jnp.float32
    c_wr, c_wu, memw, read, cmin, carg = pl.pallas_call(
        _pass1_body,
        grid=(NB,),
        in_specs=[
            pl.BlockSpec((MB, U), lambda i: (i, 0)),
            pl.BlockSpec((MB, B), lambda i: (i, 0)),
            pl.BlockSpec((MB, B), lambda i: (i, 0)),
            pl.BlockSpec((MB, B), lambda i: (i, 0)),
            pl.BlockSpec((B, U), lambda i: (0, 0)),
            pl.BlockSpec((B, U), lambda i: (0, 0)),
            pl.BlockSpec((8, 128), lambda i: (0, 0)),
        ],
        out_specs=[
            pl.BlockSpec((MB, B), lambda i: (i, 0)),
            pl.BlockSpec((MB, B), lambda i: (i, 0)),
            pl.BlockSpec((MB, U), lambda i: (i, 0)),
            pl.BlockSpec((B, U), lambda i: (0, 0)),
            pl.BlockSpec((8, B), lambda i: (0, 0)),
            pl.BlockSpec((8, B), lambda i: (0, 0)),
        ],
        out_shape=[
            jax.ShapeDtypeStruct((M, B), f32),
            jax.ShapeDtypeStruct((M, B), f32),
            jax.ShapeDtypeStruct((M, U), f32),
            jax.ShapeDtypeStruct((B, U), f32),
            jax.ShapeDtypeStruct((8, B), f32),
            jax.ShapeDtypeStruct((8, B), jnp.int32),
        ],
    )(m_tm1, c_wu_tm1, c_wlu_tm1, c_wr_tm1, n_key, key_list, wg8)

    c_wlu, memory = pl.pallas_call(
        _pass2_body,
        grid=(NB,),
        in_specs=[
            pl.BlockSpec((MB, B), lambda i: (i, 0)),
            pl.BlockSpec((8, B), lambda i: (0, 0)),
            pl.BlockSpec((8, B), lambda i: (0, 0)),
            pl.BlockSpec((MB, U), lambda i: (i, 0)),
            pl.BlockSpec((MB, U), lambda i: (i, 0)),
        ],
        out_specs=[
            pl.BlockSpec((MB, B), lambda i: (i, 0)),
            pl.BlockSpec((MB, U), lambda i: (i, 0)),
        ],
        out_shape=[
            jax.ShapeDtypeStruct((M, B), f32),
            jax.ShapeDtypeStruct((M, U), f32),
        ],
        scratch_shapes=[pltpu.SMEM((1,), jnp.int32)],
    )(c_wu, cmin, carg, memw, m_tm1)

    return (read, read, memory, c_wu, c_wlu, c_wr, key_list, c_ctrl_new)


# speedup vs baseline: 30.9755x; 1.0365x over previous
import jax
import jax.numpy as jnp
from jax.experimental import pallas as pl
from jax.experimental.pallas import tpu as pltpu

B, D, U, M = 1024, 128, 64, 16384
USAGE_DECAY = 0.95
MB = 1024
NB = M // MB


def _lstm_body(ctrl_ref, h_ref, c_ref, wk_ref, uk_ref, bk_ref,
               key_ref, cnew_ref, nkey_ref):
    z = (jax.lax.dot_general(ctrl_ref[...], wk_ref[...],
                             (((1,), (0,)), ((), ())),
                             preferred_element_type=jnp.float32)
         + jax.lax.dot_general(h_ref[...], uk_ref[...],
                               (((1,), (0,)), ((), ())),
                               preferred_element_type=jnp.float32)
         + bk_ref[0:1, :])
    zi = z[:, 0 * U:1 * U]
    zf = z[:, 1 * U:2 * U]
    zc = z[:, 2 * U:3 * U]
    zo = z[:, 3 * U:4 * U]
    i = jax.nn.sigmoid(zi)
    f = jax.nn.sigmoid(zf)
    c_new = f * c_ref[...] + i * jnp.tanh(zc)
    o = jax.nn.sigmoid(zo)
    key = o * jnp.tanh(c_new)
    key_ref[...] = key
    cnew_ref[...] = c_new
    nkey_ref[...] = key / jnp.sqrt(
        jnp.maximum(jnp.sum(key * key, axis=1, keepdims=True), 1e-12))


def _pass1_body(m_ref, cwu1_ref, cwlu1_ref, cwr1_ref, nkey_ref, key_ref,
                wg_ref,
                cwr_ref, cwu_ref, memw_ref, read_ref, cmin_ref, carg_ref):
    step = pl.program_id(0)
    mblk = m_ref[...]
    nm = mblk / jnp.sqrt(
        jnp.maximum(jnp.sum(mblk * mblk, axis=1, keepdims=True), 1e-12))
    cos = jax.lax.dot_general(nm, nkey_ref[...],
                              (((1,), (1,)), ((), ())),
                              preferred_element_type=jnp.float32)
    rowmax = jnp.max(cos, axis=1, keepdims=True)
    e = jnp.exp(cos - rowmax)
    cwr = e / jnp.sum(e, axis=1, keepdims=True)
    cwr_ref[...] = cwr

    wg = wg_ref[0, 0]
    cww = wg * cwr1_ref[...] + (1.0 - wg) + cwlu1_ref[...]
    cwu = USAGE_DECAY * cwu1_ref[...] + cwr + cww
    cwu_ref[...] = cwu

    memw_ref[...] = jax.lax.dot_general(cww, key_ref[...],
                                        (((1,), (0,)), ((), ())),
                                        preferred_element_type=jnp.float32)

    rpart = jax.lax.dot_general(cwr, mblk,
                                (((0,), (0,)), ((), ())),
                                preferred_element_type=jnp.float32)

    blkmin = jnp.min(cwu, axis=0, keepdims=True)
    rows = jax.lax.broadcasted_iota(jnp.int32, (MB, B), 0) + step * MB
    cand = jnp.max(jnp.where(cwu == blkmin, rows, -1),
                   axis=0, keepdims=True)
    blkmin8 = jnp.broadcast_to(blkmin, (8, B))
    cand8 = jnp.broadcast_to(cand, (8, B))

    @pl.when(step == 0)
    def _init():
        read_ref[...] = rpart
        cmin_ref[...] = blkmin8
        carg_ref[...] = cand8

    @pl.when(step != 0)
    def _acc():
        read_ref[...] += rpart
        run_min = cmin_ref[...]
        upd = blkmin8 <= run_min
        carg_ref[...] = jnp.where(upd, cand8, carg_ref[...])
        cmin_ref[...] = jnp.minimum(run_min, blkmin8)


def _pass2_body(cwu_ref, cmin_ref, carg_ref, memw_ref, m_ref,
                cwlu_ref, mem_ref, sel_ref):
    step = pl.program_id(0)

    @pl.when(step == 0)
    def _select():
        cm = cmin_ref[0:1, :]
        minv = jnp.min(cm)
        lane = jax.lax.broadcasted_iota(jnp.int32, (1, B), 1)
        i_nth = jnp.min(jnp.where(cm == minv, lane, 2 ** 30))
        selv = jnp.max(jnp.where(lane == i_nth, carg_ref[0:1, :], -1))
        sel_ref[0] = selv

    cmrow = cmin_ref[0:1, :]
    cwlu_ref[...] = (cwu_ref[...] <= cmrow).astype(jnp.float32)

    sel = sel_ref[0]
    rows = jax.lax.broadcasted_iota(jnp.int32, (MB, 1), 0) + step * MB
    keep = (rows != sel).astype(jnp.float32)
    mem_ref[...] = memw_ref[...] + (keep * float(B)) * m_ref[...]


def kernel(inputs, r_tm1, m_tm1, c_wu_tm1, c_wlu_tm1, c_wr_tm1, h_tm1,
           c_tm1, write_gate, Wk, Uk, bk):
    ctrl_in = jnp.concatenate([inputs, r_tm1], axis=1)
    bk8 = jnp.broadcast_to(bk.reshape(1, 4 * U), (8, 4 * U))
    wg8 = jnp.broadcast_to(jax.nn.sigmoid(write_gate).reshape(1, 1), (8, 128))

    key_list, c_ctrl_new, n_key = pl.pallas_call(
        _lstm_body,
        out_shape=[jax.ShapeDtypeStruct((B, U), jnp.float32)] * 3,
    )(ctrl_in, h_tm1, c_tm1, Wk, Uk, bk8)

    f32 = jnp.float32
    c_wr, c_wu, memw, read, cmin, carg = pl.pallas_call(
        _pass1_body,
        grid=(NB,),
        in_specs=[
            pl.BlockSpec((MB, U), lambda i: (i, 0)),
            pl.BlockSpec((MB, B), lambda i: (i, 0)),
            pl.BlockSpec((MB, B), lambda i: (i, 0)),
            pl.BlockSpec((MB, B), lambda i: (i, 0)),
            pl.BlockSpec((B, U), lambda i: (0, 0)),
            pl.BlockSpec((B, U), lambda i: (0, 0)),
            pl.BlockSpec((8, 128), lambda i: (0, 0)),
        ],
        out_specs=[
            pl.BlockSpec((MB, B), lambda i: (i, 0)),
            pl.BlockSpec((MB, B), lambda i: (i, 0)),
            pl.BlockSpec((MB, U), lambda i: (i, 0)),
            pl.BlockSpec((B, U), lambda i: (0, 0)),
            pl.BlockSpec((8, B), lambda i: (0, 0)),
            pl.BlockSpec((8, B), lambda i: (0, 0)),
        ],
        out_shape=[
            jax.ShapeDtypeStruct((M, B), f32),
            jax.ShapeDtypeStruct((M, B), f32),
            jax.ShapeDtypeStruct((M, U), f32),
            jax.ShapeDtypeStruct((B, U), f32),
            jax.ShapeDtypeStruct((8, B), f32),
            jax.ShapeDtypeStruct((8, B), jnp.int32),
        ],
    )(m_tm1, c_wu_tm1, c_wlu_tm1, c_wr_tm1, n_key, key_list, wg8)

    c_wlu, memory = pl.pallas_call(
        _pass2_body,
        grid=(NB,),
        in_specs=[
            pl.BlockSpec((MB, B), lambda i: (i, 0)),
            pl.BlockSpec((8, B), lambda i: (0, 0)),
            pl.BlockSpec((8, B), lambda i: (0, 0)),
            pl.BlockSpec((MB, U), lambda i: (i, 0)),
            pl.BlockSpec((MB, U), lambda i: (i, 0)),
        ],
        out_specs=[
            pl.BlockSpec((MB, B), lambda i: (i, 0)),
            pl.BlockSpec((MB, U), lambda i: (i, 0)),
        ],
        out_shape=[
            jax.ShapeDtypeStruct((M, B), f32),
            jax.ShapeDtypeStruct((M, U), f32),
        ],
        scratch_shapes=[pltpu.SMEM((1,), jnp.int32)],
    )(c_wu, cmin, carg, memw, m_tm1)

    return (read, read, memory, c_wu, c_wlu, c_wr, key_list, c_ctrl_new)


# speedup vs baseline: 33.0870x; 1.0682x over previous
import jax
import jax.numpy as jnp
from jax.experimental import pallas as pl
from jax.experimental.pallas import tpu as pltpu

B, D, U, M = 1024, 128, 64, 16384
USAGE_DECAY = 0.95
MB = 512
NB = M // MB


def _body(ctrl_ref, h_ref, c_ref, wk_ref, uk_ref, bk_ref, wg_ref,
          m_ref, cwu1_ref, cwlu1_ref, cwr1_ref,
          key_ref, cnew_ref, read_ref, cwr_ref, cwu_ref, cmin_ref,
          carg_ref, cwlu_ref, mem_ref,
          nkey_s, memw_s, lmask_s, blkmin_s, sel_s):
    p = pl.program_id(0)
    i = pl.program_id(1)

    @pl.when(jnp.logical_and(p == 0, i == 0))
    def _lstm():
        z = (jax.lax.dot_general(ctrl_ref[...], wk_ref[...],
                                 (((1,), (0,)), ((), ())),
                                 preferred_element_type=jnp.float32)
             + jax.lax.dot_general(h_ref[...], uk_ref[...],
                                   (((1,), (0,)), ((), ())),
                                   preferred_element_type=jnp.float32)
             + bk_ref[0:1, :])
        gi = jax.nn.sigmoid(z[:, 0 * U:1 * U])
        gf = jax.nn.sigmoid(z[:, 1 * U:2 * U])
        c_new = gf * c_ref[...] + gi * jnp.tanh(z[:, 2 * U:3 * U])
        go = jax.nn.sigmoid(z[:, 3 * U:4 * U])
        key = go * jnp.tanh(c_new)
        key_ref[...] = key
        cnew_ref[...] = c_new
        nkey_s[...] = key / jnp.sqrt(
            jnp.maximum(jnp.sum(key * key, axis=1, keepdims=True), 1e-12))

    @pl.when(p == 0)
    def _phase0():
        mblk = m_ref[...]
        nm = mblk / jnp.sqrt(
            jnp.maximum(jnp.sum(mblk * mblk, axis=1, keepdims=True), 1e-12))
        cos = jax.lax.dot_general(nm, nkey_s[...],
                                  (((1,), (1,)), ((), ())),
                                  preferred_element_type=jnp.float32)
        rowmax = jnp.max(cos, axis=1, keepdims=True)
        e = jnp.exp(cos - rowmax)
        cwr = e / jnp.sum(e, axis=1, keepdims=True)
        cwr_ref[...] = cwr

        wg = wg_ref[0, 0]
        cww = wg * cwr1_ref[...] + (1.0 - wg) + cwlu1_ref[...]
        cwu = USAGE_DECAY * cwu1_ref[...] + cwr + cww
        cwu_ref[...] = cwu

        memw_s[pl.ds(i * MB, MB), :] = jax.lax.dot_general(
            cww, key_ref[...], (((1,), (0,)), ((), ())),
            preferred_element_type=jnp.float32)

        rpart = jax.lax.dot_general(cwr, mblk,
                                    (((0,), (0,)), ((), ())),
                                    preferred_element_type=jnp.float32)

        blkmin = jnp.min(cwu, axis=0, keepdims=True)
        is_lmin = cwu == blkmin
        lmask_s[pl.ds(i * MB, MB), :] = is_lmin.astype(jnp.int8)
        blkmin_s[pl.ds(i, 1), :] = blkmin
        rows = jax.lax.broadcasted_iota(jnp.int32, (MB, B), 0) + i * MB
        cand = jnp.max(jnp.where(is_lmin, rows, -1),
                       axis=0, keepdims=True)
        blkmin8 = jnp.broadcast_to(blkmin, (8, B))
        cand8 = jnp.broadcast_to(cand, (8, B))

        @pl.when(i == 0)
        def _init():
            read_ref[...] = rpart
            cmin_ref[...] = blkmin8
            carg_ref[...] = cand8

        @pl.when(i != 0)
        def _acc():
            read_ref[...] += rpart
            run_min = cmin_ref[...]
            upd = blkmin8 <= run_min
            carg_ref[...] = jnp.where(upd, cand8, carg_ref[...])
            cmin_ref[...] = jnp.minimum(run_min, blkmin8)

    @pl.when(p == 1)
    def _phase1():
        @pl.when(i == 0)
        def _select():
            cm = cmin_ref[0:1, :]
            minv = jnp.min(cm)
            lane = jax.lax.broadcasted_iota(jnp.int32, (1, B), 1)
            i_nth = jnp.min(jnp.where(cm == minv, lane, 2 ** 30))
            selv = jnp.max(jnp.where(lane == i_nth, carg_ref[0:1, :], -1))
            sel_s[0] = selv

        lmin = lmask_s[pl.ds(i * MB, MB), :].astype(jnp.float32)
        gmin = (blkmin_s[pl.ds(i, 1), :]
                == cmin_ref[0:1, :]).astype(jnp.float32)
        cwlu_ref[...] = lmin * gmin

        sel = sel_s[0]
        rows = jax.lax.broadcasted_iota(jnp.int32, (MB, 1), 0) + i * MB
        keep = (rows != sel).astype(jnp.float32)
        mem_ref[...] = (memw_s[pl.ds(i * MB, MB), :]
                        + (keep * float(B)) * m_ref[...])


def kernel(inputs, r_tm1, m_tm1, c_wu_tm1, c_wlu_tm1, c_wr_tm1, h_tm1,
           c_tm1, write_gate, Wk, Uk, bk):
    ctrl_in = jnp.concatenate([inputs, r_tm1], axis=1)
    bk8 = jnp.broadcast_to(bk.reshape(1, 4 * U), (8, 4 * U))
    wg8 = jnp.broadcast_to(jax.nn.sigmoid(write_gate).reshape(1, 1), (8, 128))
    f32 = jnp.float32

    fixed = lambda p, i: (0, 0)
    blk_p0 = lambda p, i: (i * (1 - p) + (NB - 1) * p, 0)
    blk_p1 = lambda p, i: (i * p, 0)
    blk_both = lambda p, i: (i, 0)

    (key_list, c_ctrl_new, read, c_wr, c_wu, cmin, carg, c_wlu,
     memory) = pl.pallas_call(
        _body,
        grid=(2, NB),
        in_specs=[
            pl.BlockSpec((B, D + U), fixed),
            pl.BlockSpec((B, U), fixed),
            pl.BlockSpec((B, U), fixed),
            pl.BlockSpec((D + U, 4 * U), fixed),
            pl.BlockSpec((U, 4 * U), fixed),
            pl.BlockSpec((8, 4 * U), fixed),
            pl.BlockSpec((8, 128), fixed),
            pl.BlockSpec((MB, U), blk_both),
            pl.BlockSpec((MB, B), blk_p0),
            pl.BlockSpec((MB, B), blk_p0),
            pl.BlockSpec((MB, B), blk_p0),
        ],
        out_specs=[
            pl.BlockSpec((B, U), fixed),
            pl.BlockSpec((B, U), fixed),
            pl.BlockSpec((B, U), fixed),
            pl.BlockSpec((MB, B), blk_p0),
            pl.BlockSpec((MB, B), blk_p0),
            pl.BlockSpec((8, B), fixed),
            pl.BlockSpec((8, B), fixed),
            pl.BlockSpec((MB, B), blk_p1),
            pl.BlockSpec((MB, U), blk_p1),
        ],
        out_shape=[
            jax.ShapeDtypeStruct((B, U), f32),
            jax.ShapeDtypeStruct((B, U), f32),
            jax.ShapeDtypeStruct((B, U), f32),
            jax.ShapeDtypeStruct((M, B), f32),
            jax.ShapeDtypeStruct((M, B), f32),
            jax.ShapeDtypeStruct((8, B), f32),
            jax.ShapeDtypeStruct((8, B), jnp.int32),
            jax.ShapeDtypeStruct((M, B), f32),
            jax.ShapeDtypeStruct((M, U), f32),
        ],
        scratch_shapes=[
            pltpu.VMEM((B, U), f32),
            pltpu.VMEM((M, U), f32),
            pltpu.VMEM((M, B), jnp.int8),
            pltpu.VMEM((NB, B), f32),
            pltpu.SMEM((1,), jnp.int32),
        ],
    )(ctrl_in, h_tm1, c_tm1, Wk, Uk, bk8, wg8,
      m_tm1, c_wu_tm1, c_wlu_tm1, c_wr_tm1)

    return (read, read, memory, c_wu, c_wlu, c_wr, key_list, c_ctrl_new)


# speedup vs baseline: 33.4206x; 1.0101x over previous
import jax
import jax.numpy as jnp
from jax.experimental import pallas as pl
from jax.experimental.pallas import tpu as pltpu

B, D, U, M = 1024, 128, 64, 16384
USAGE_DECAY = 0.95
MB = 512
NB = M // MB


def _body(ctrl_ref, h_ref, c_ref, wk_ref, uk_ref, bk_ref, wg_ref,
          m_ref, cwu1_ref, cwlu1_ref, cwr1_ref,
          key_ref, cnew_ref, read_ref, cwr_ref, cwu_ref, cwlu_ref, mem_ref,
          nkey_s, memw_s, lmask_s, blkmin_s, cmin_s, sel_s):
    p = pl.program_id(0)
    i = pl.program_id(1)

    @pl.when(jnp.logical_and(p == 0, i == 0))
    def _lstm():
        z = (jax.lax.dot_general(ctrl_ref[...], wk_ref[...],
                                 (((1,), (0,)), ((), ())),
                                 preferred_element_type=jnp.float32)
             + jax.lax.dot_general(h_ref[...], uk_ref[...],
                                   (((1,), (0,)), ((), ())),
                                   preferred_element_type=jnp.float32)
             + bk_ref[0:1, :])
        gi = jax.nn.sigmoid(z[:, 0 * U:1 * U])
        gf = jax.nn.sigmoid(z[:, 1 * U:2 * U])
        c_new = gf * c_ref[...] + gi * jnp.tanh(z[:, 2 * U:3 * U])
        go = jax.nn.sigmoid(z[:, 3 * U:4 * U])
        key = go * jnp.tanh(c_new)
        key_ref[...] = key
        cnew_ref[...] = c_new
        nkey_s[...] = key / jnp.sqrt(
            jnp.maximum(jnp.sum(key * key, axis=1, keepdims=True), 1e-12))

    @pl.when(p == 0)
    def _phase0():
        mblk = m_ref[...]
        nm = mblk / jnp.sqrt(
            jnp.maximum(jnp.sum(mblk * mblk, axis=1, keepdims=True), 1e-12))
        cos = jax.lax.dot_general(nm, nkey_s[...],
                                  (((1,), (1,)), ((), ())),
                                  preferred_element_type=jnp.float32)
        e = jnp.exp(cos)
        cwr = e / jnp.sum(e, axis=1, keepdims=True)
        cwr_ref[...] = cwr

        wg = wg_ref[0, 0]
        cww = wg * cwr1_ref[...] + (1.0 - wg) + cwlu1_ref[...]
        cwu = USAGE_DECAY * cwu1_ref[...] + cwr + cww
        cwu_ref[...] = cwu

        memw_s[pl.ds(i * MB, MB), :] = jax.lax.dot_general(
            cww, key_ref[...], (((1,), (0,)), ((), ())),
            preferred_element_type=jnp.float32)

        rpart = jax.lax.dot_general(cwr, mblk,
                                    (((0,), (0,)), ((), ())),
                                    preferred_element_type=jnp.float32)

        blkmin = jnp.min(cwu, axis=0, keepdims=True)
        lmask_s[pl.ds(i * MB, MB), :] = (cwu == blkmin).astype(jnp.int8)
        blkmin_s[pl.ds(i, 1), :] = blkmin

        @pl.when(i == 0)
        def _init():
            read_ref[...] = rpart

        @pl.when(i != 0)
        def _acc():
            read_ref[...] += rpart

    @pl.when(p == 1)
    def _phase1():
        @pl.when(i == 0)
        def _select():
            bm = blkmin_s[...]
            cm = jnp.min(bm, axis=0, keepdims=True)
            cmin_s[...] = cm
            minv = jnp.min(cm)
            lane = jax.lax.broadcasted_iota(jnp.int32, (1, B), 1)
            i_nth = jnp.min(jnp.where(cm == minv, lane, 2 ** 30))
            colf = (lane == i_nth).astype(jnp.float32)
            blks = jax.lax.broadcasted_iota(jnp.int32, (NB, B), 0)
            hit = (bm == cm).astype(jnp.float32) * colf
            bsel = jnp.max(jnp.where(hit > 0.0, blks, -1))
            lblk = lmask_s[pl.ds(bsel * MB, MB), :].astype(jnp.float32)
            rows = jax.lax.broadcasted_iota(jnp.int32, (MB, B), 0)
            rsel = jnp.max(jnp.where(lblk * colf > 0.0, rows, -1))
            sel_s[0] = bsel * MB + rsel

        lmin = lmask_s[pl.ds(i * MB, MB), :].astype(jnp.float32)
        gmin = (blkmin_s[pl.ds(i, 1), :] == cmin_s[...]).astype(jnp.float32)
        cwlu_ref[...] = lmin * gmin

        sel = sel_s[0]
        rows = jax.lax.broadcasted_iota(jnp.int32, (MB, 1), 0) + i * MB
        keep = (rows != sel).astype(jnp.float32)
        mem_ref[...] = (memw_s[pl.ds(i * MB, MB), :]
                        + (keep * float(B)) * m_ref[...])


def kernel(inputs, r_tm1, m_tm1, c_wu_tm1, c_wlu_tm1, c_wr_tm1, h_tm1,
           c_tm1, write_gate, Wk, Uk, bk):
    ctrl_in = jnp.concatenate([inputs, r_tm1], axis=1)
    bk8 = jnp.broadcast_to(bk.reshape(1, 4 * U), (8, 4 * U))
    wg8 = jnp.broadcast_to(jax.nn.sigmoid(write_gate).reshape(1, 1), (8, 128))
    f32 = jnp.float32

    fixed = lambda p, i: (0, 0)
    blk_p0 = lambda p, i: (i * (1 - p) + (NB - 1) * p, 0)
    blk_p1 = lambda p, i: (i * p, 0)
    blk_both = lambda p, i: (i, 0)

    (key_list, c_ctrl_new, read, c_wr, c_wu, c_wlu, memory) = pl.pallas_call(
        _body,
        grid=(2, NB),
        in_specs=[
            pl.BlockSpec((B, D + U), fixed),
            pl.BlockSpec((B, U), fixed),
            pl.BlockSpec((B, U), fixed),
            pl.BlockSpec((D + U, 4 * U), fixed),
            pl.BlockSpec((U, 4 * U), fixed),
            pl.BlockSpec((8, 4 * U), fixed),
            pl.BlockSpec((8, 128), fixed),
            pl.BlockSpec((MB, U), blk_both),
            pl.BlockSpec((MB, B), blk_p0),
            pl.BlockSpec((MB, B), blk_p0),
            pl.BlockSpec((MB, B), blk_p0),
        ],
        out_specs=[
            pl.BlockSpec((B, U), fixed),
            pl.BlockSpec((B, U), fixed),
            pl.BlockSpec((B, U), fixed),
            pl.BlockSpec((MB, B), blk_p0),
            pl.BlockSpec((MB, B), blk_p0),
            pl.BlockSpec((MB, B), blk_p1),
            pl.BlockSpec((MB, U), blk_p1),
        ],
        out_shape=[
            jax.ShapeDtypeStruct((B, U), f32),
            jax.ShapeDtypeStruct((B, U), f32),
            jax.ShapeDtypeStruct((B, U), f32),
            jax.ShapeDtypeStruct((M, B), f32),
            jax.ShapeDtypeStruct((M, B), f32),
            jax.ShapeDtypeStruct((M, B), f32),
            jax.ShapeDtypeStruct((M, U), f32),
        ],
        scratch_shapes=[
            pltpu.VMEM((B, U), f32),
            pltpu.VMEM((M, U), f32),
            pltpu.VMEM((M, B), jnp.int8),
            pltpu.VMEM((NB, B), f32),
            pltpu.VMEM((1, B), f32),
            pltpu.SMEM((1,), jnp.int32),
        ],
    )(ctrl_in, h_tm1, c_tm1, Wk, Uk, bk8, wg8,
      m_tm1, c_wu_tm1, c_wlu_tm1, c_wr_tm1)

    return (read, read, memory, c_wu, c_wlu, c_wr, key_list, c_ctrl_new)
